# Initial kernel scaffold; baseline (speedup 1.0000x reference)
import jax
import jax.numpy as jnp
from jax.experimental import pallas as pl


def kernel(encoder_output, mask, W, b):
    raise NotImplementedError("write your pallas kernel here")


# speedup vs baseline: 58.5262x; 58.5262x over previous
import jax
import jax.numpy as jnp
from jax import lax
from jax.experimental import pallas as pl


def _loss_kernel(enc_ref, mask_ref, w_ref, b_ref, out_ref):
    B, L, E = enc_ref.shape
    enc = enc_ref[...]
    msk = mask_ref[...]
    W = w_ref[...]
    b = b_ref[...]

    turn_lengths = jnp.sum(msk, axis=1)
    pos = lax.broadcasted_iota(jnp.int32, (B, L), 1).astype(jnp.float32)
    valid = (pos < turn_lengths[:, None]).astype(enc.dtype)
    his = enc * valid[:, :, None]

    hflat = his.reshape(B * L, E)
    a0 = jnp.sum(hflat * W[0, :E][None, :], axis=1).reshape(B, L) + b[0]
    a1 = jnp.sum(hflat * W[1, :E][None, :], axis=1).reshape(B, L) + b[1]
    c0 = jnp.sum(hflat * W[0, E:][None, :], axis=1).reshape(B, L)
    c1 = jnp.sum(hflat * W[1, E:][None, :], axis=1).reshape(B, L)

    x0 = a0[:, :, None] + c0[:, None, :]
    x1 = a1[:, :, None] + c1[:, None, :]
    m = jnp.maximum(x0, x1)
    lse = m + jnp.log1p(jnp.exp(-jnp.abs(x1 - x0)))

    jj = lax.broadcasted_iota(jnp.int32, (L, L), 0)
    kk = lax.broadcasted_iota(jnp.int32, (L, L), 1)
    tri = (kk < jj)[None, :, :]
    pick = jnp.where((kk == jj - 1)[None, :, :], x1, x0)
    total = jnp.sum(jnp.where(tri, pick - lse, 0.0))

    n_pairs = B * (L * (L - 1)) // 2
    out_ref[...] = jnp.reshape(-total / n_pairs, (1, 1))


def kernel(encoder_output, mask, W, b):
    out = pl.pallas_call(
        _loss_kernel,
        out_shape=jax.ShapeDtypeStruct((1, 1), jnp.float32),
    )(encoder_output, mask, W, b)
    return out[0, 0]
